# baseline (speedup 1.0000x reference)
import jax
import jax.numpy as jnp
from jax.experimental import pallas as pl
from jax.experimental.pallas import tpu as pltpu

B, N = 32, 36864
BN = B * N
OBJ_ROWS = BN // 128
BOX_ROWS = (BN * 4) // 128
MASK_COLS = 32
GRID = 36
OBJ_BLK = OBJ_ROWS // GRID
BOX_BLK = BOX_ROWS // GRID


def _body(lam_ref, op_ref, gt_ref, bp_ref, bg_ref, gtm_ref,
          o1_ref, o2_ref, o3_ref, acc_ref):
    i = pl.program_id(0)

    x = op_ref[...]
    t = gt_ref[...]
    per = jnp.maximum(x, 0.0) - x * t + jnp.log1p(jnp.exp(-jnp.abs(x)))
    s_bce = jnp.sum(per)

    d = bp_ref[...] - bg_ref[...]
    a = jnp.abs(d)
    c = jnp.minimum(a, 1.0)
    per2 = c * (a + a - c)
    lane = jax.lax.broadcasted_iota(jnp.int32, (128, MASK_COLS), 0)
    grp = jax.lax.broadcasted_iota(jnp.int32, (128, MASK_COLS), 1)
    fold = (lane // 4 == grp).astype(jnp.float32)
    folded = jax.lax.dot(per2, fold, preferred_element_type=jnp.float32)
    mask = (gtm_ref[...] == 1.0).astype(jnp.float32)
    s_box = jnp.sum(folded * mask)

    @pl.when(i == 0)
    def _():
        acc_ref[0] = s_bce
        acc_ref[1] = s_box

    @pl.when(i > 0)
    def _():
        acc_ref[0] += s_bce
        acc_ref[1] += s_box

    @pl.when(i == GRID - 1)
    def _():
        lam_o = lam_ref[0]
        lam_b = lam_ref[1]
        o1 = lam_o * acc_ref[0] * (1.0 / BN)
        o2 = lam_b * (0.5 * acc_ref[1])
        o1_ref[...] = jnp.broadcast_to(o1, (1, 1))
        o2_ref[...] = jnp.broadcast_to(o2, (1, 1))
        o3_ref[...] = jnp.broadcast_to(o1 + o2, (1, 1))


def kernel(objectness_pred, bbox_pred, objectness_gt, bbox_gt,
           lambda_rpn_objectness, lambda_rpn_bbox):
    op2 = objectness_pred.reshape(OBJ_ROWS, 128)
    gt2 = objectness_gt.reshape(OBJ_ROWS, 128)
    gtm = objectness_gt.reshape(BOX_ROWS, MASK_COLS)
    bp2 = bbox_pred.reshape(BOX_ROWS, 128)
    bg2 = bbox_gt.reshape(BOX_ROWS, 128)
    lam = jnp.stack([jnp.asarray(lambda_rpn_objectness, jnp.float32),
                     jnp.asarray(lambda_rpn_bbox, jnp.float32)])

    out_shape = [jax.ShapeDtypeStruct((1, 1), jnp.float32)] * 3
    grid_spec = pltpu.PrefetchScalarGridSpec(
        num_scalar_prefetch=1,
        grid=(GRID,),
        in_specs=[
            pl.BlockSpec((OBJ_BLK, 128), lambda i, *_: (i, 0)),
            pl.BlockSpec((OBJ_BLK, 128), lambda i, *_: (i, 0)),
            pl.BlockSpec((BOX_BLK, 128), lambda i, *_: (i, 0)),
            pl.BlockSpec((BOX_BLK, 128), lambda i, *_: (i, 0)),
            pl.BlockSpec((BOX_BLK, MASK_COLS), lambda i, *_: (i, 0)),
        ],
        out_specs=[pl.BlockSpec((1, 1), lambda i, *_: (0, 0))] * 3,
        scratch_shapes=[pltpu.SMEM((2,), jnp.float32)],
    )
    o1, o2, o3 = pl.pallas_call(
        _body,
        grid_spec=grid_spec,
        out_shape=out_shape,
    )(lam, op2, gt2, bp2, bg2, gtm)
    return (o1.reshape(()), o2.reshape(()), o3.reshape(()))


# speedup vs baseline: 60.2448x; 60.2448x over previous
import functools

import jax
import jax.numpy as jnp
from jax import lax
from jax.experimental import pallas as pl
from jax.experimental.pallas import tpu as pltpu
from jax.experimental.pallas import tpu_sc as plsc

B, N = 32, 36864
BN = B * N
KBLK = N // 128
ROWS = B * KBLK * 4

BCE_GRID = 36
BCE_CHUNK = N // BCE_GRID


def _bce_body(op_ref, gt_ref, out_ref):
    i = pl.program_id(0)
    x = op_ref[...]
    t = gt_ref[...]
    per = jnp.maximum(x, 0.0) - x * t + jnp.log1p(jnp.exp(-jnp.abs(x)))

    @pl.when(i == 0)
    def _():
        out_ref[...] = jnp.zeros_like(out_ref)

    out_ref[...] += jnp.broadcast_to(jnp.sum(per), (1, 1))


def _bce_sum(op, gt):
    return pl.pallas_call(
        _bce_body,
        grid=(BCE_GRID,),
        in_specs=[
            pl.BlockSpec((B, BCE_CHUNK), lambda i: (0, i)),
            pl.BlockSpec((B, BCE_CHUNK), lambda i: (0, i)),
        ],
        out_specs=pl.BlockSpec((1, 1), lambda i: (0, 0)),
        out_shape=jax.ShapeDtypeStruct((1, 1), jnp.float32),
    )(op, gt)


NW = 32
CHUNK_ROWS = 128
N_CHUNKS = (KBLK * 4) // CHUNK_ROWS

_sc_mesh = plsc.VectorSubcoreMesh(core_axis_name="c", subcore_axis_name="s")


@functools.partial(
    pl.kernel,
    mesh=_sc_mesh,
    out_type=jax.ShapeDtypeStruct((NW, 16), jnp.float32),
    scratch_types=[
        pltpu.VMEM((KBLK, 128), jnp.float32),
        pltpu.VMEM((2, CHUNK_ROWS, 128), jnp.float32),
        pltpu.VMEM((2, CHUNK_ROWS, 128), jnp.float32),
        pltpu.VMEM((16,), jnp.float32),
        pltpu.SemaphoreType.DMA,
        pltpu.SemaphoreType.DMA,
        pltpu.SemaphoreType.DMA,
    ],
)
def _sc_bbox(bp_hbm, bg_hbm, gt4_hbm, out_hbm,
             mask_v, bp_v, bg_v, acc_v, sem_m, sem_p, sem_g):
    cid = lax.axis_index("c")
    sid = lax.axis_index("s")
    w = sid * 2 + cid
    bb = w // 8
    bi = w % 8

    pltpu.sync_copy(gt4_hbm.at[bb, :, bi, :], mask_v)

    row0 = w * (KBLK * 4)

    def chunk_start(step, buf):
        r = row0 + step * CHUNK_ROWS
        cp = pltpu.async_copy(bp_hbm.at[pl.ds(r, CHUNK_ROWS), :],
                              bp_v.at[buf], sem_p)
        cg = pltpu.async_copy(bg_hbm.at[pl.ds(r, CHUNK_ROWS), :],
                              bg_v.at[buf], sem_g)
        return cp, cg

    def chunk_sum(step, buf):
        k0 = step * (CHUNK_ROWS // 4)

        def kk_body(kk, acc):
            def a_body(ai, acc2):
                a0 = ai * 16
                m = mask_v[k0 + kk, pl.ds(a0, 16)]

                def c_body(c, acc3):
                    r = kk * 4 + c
                    p = bp_v[buf, r, pl.ds(a0, 16)]
                    g = bg_v[buf, r, pl.ds(a0, 16)]
                    d = p - g
                    a = jnp.abs(d)
                    cc = jnp.minimum(a, 1.0)
                    per2 = cc * (a + a - cc)
                    return acc3 + per2 * m

                return lax.fori_loop(0, 4, c_body, acc2)

            return lax.fori_loop(0, 8, a_body, acc)

        return lax.fori_loop(0, CHUNK_ROWS // 4, kk_body,
                             jnp.zeros((16,), jnp.float32))

    total = jnp.zeros((16,), jnp.float32)
    cp, cg = chunk_start(0, 0)
    for step in range(N_CHUNKS):
        cp.wait()
        cg.wait()
        if step + 1 < N_CHUNKS:
            cp, cg = chunk_start(step + 1, (step + 1) % 2)
        total = total + chunk_sum(step, step % 2)

    acc_v[...] = total * 0.5
    pltpu.sync_copy(acc_v, out_hbm.at[w])


def _combine_body(bce_ref, part_ref, lam_ref, o1_ref, o2_ref, o3_ref):
    lam_o = lam_ref[0, 0]
    lam_b = lam_ref[0, 1]
    o1 = lam_o * bce_ref[0, 0] * (1.0 / BN)
    o2 = lam_b * jnp.sum(part_ref[...])
    o1_ref[...] = jnp.broadcast_to(o1, (1, 1))
    o2_ref[...] = jnp.broadcast_to(o2, (1, 1))
    o3_ref[...] = jnp.broadcast_to(o1 + o2, (1, 1))


def _combine(bce, parts, lam):
    return pl.pallas_call(
        _combine_body,
        in_specs=[
            pl.BlockSpec(memory_space=pltpu.SMEM),
            pl.BlockSpec((NW, 16), lambda: (0, 0)),
            pl.BlockSpec(memory_space=pltpu.SMEM),
        ],
        out_specs=[pl.BlockSpec((1, 1), lambda: (0, 0))] * 3,
        out_shape=[jax.ShapeDtypeStruct((1, 1), jnp.float32)] * 3,
    )(bce, parts, lam)


def kernel(objectness_pred, bbox_pred, objectness_gt, bbox_gt,
           lambda_rpn_objectness, lambda_rpn_bbox):
    bp_view = (bbox_pred.reshape(B, KBLK, 128, 4)
               .transpose(0, 1, 3, 2).reshape(ROWS, 128))
    bg_view = (bbox_gt.reshape(B, KBLK, 128, 4)
               .transpose(0, 1, 3, 2).reshape(ROWS, 128))
    gt4_view = (objectness_gt.reshape(4, 8, KBLK, 128)
                .transpose(0, 2, 1, 3))

    bce = _bce_sum(objectness_pred, objectness_gt)
    parts = _sc_bbox(bp_view, bg_view, gt4_view)
    lam = jnp.stack([jnp.asarray(lambda_rpn_objectness, jnp.float32),
                     jnp.asarray(lambda_rpn_bbox, jnp.float32)]).reshape(1, 2)
    o1, o2, o3 = _combine(bce, parts, lam)
    return (o1.reshape(()), o2.reshape(()), o3.reshape(()))


# speedup vs baseline: 70.0240x; 1.1623x over previous
import functools

import jax
import jax.numpy as jnp
from jax import lax
from jax.experimental import pallas as pl
from jax.experimental.pallas import tpu as pltpu
from jax.experimental.pallas import tpu_sc as plsc

B, N = 32, 36864
BN = B * N
KBLK = N // 128
RPB = KBLK * 4
ROWS = B * RPB

TC_FRAC = 384
TC_K = TC_FRAC // 4
OBJ_RPB = KBLK


KT = 3
KSUB = TC_K // KT
OBJ_BLK = 9216 // (4 * KT)


def _tc_body(op_ref, gt_ref, gtm_ref, *refs):
    bp_refs = refs[:8]
    bg_refs = refs[8:16]
    bce_ref, box_ref = refs[16:]
    bb = pl.program_id(0)
    t = pl.program_id(1)

    x = op_ref[...]
    tt = gt_ref[...]
    per = jnp.maximum(x, 0.0) - x * tt + jnp.log1p(jnp.exp(-jnp.abs(x)))
    s_bce = jnp.sum(per)

    maskc = (gtm_ref[...] == 1.0).astype(jnp.float32)
    fk = jax.lax.broadcasted_iota(jnp.int32, (KSUB, KSUB * 4), 0)
    fr = jax.lax.broadcasted_iota(jnp.int32, (KSUB, KSUB * 4), 1)
    fold_m = (fr // 4 == fk).astype(jnp.float32)
    dk = jax.lax.broadcasted_iota(jnp.int32, (KSUB, KSUB * 8), 0)
    dq = jax.lax.broadcasted_iota(jnp.int32, (KSUB, KSUB * 8), 1)
    s_box = jnp.zeros((), jnp.float32)
    for j in range(8):
        d = bp_refs[j][...] - bg_refs[j][...]
        a = jnp.abs(d)
        c = jnp.minimum(a, 1.0)
        per2 = c * (a + a - c)
        folded = jax.lax.dot(fold_m, per2,
                             preferred_element_type=jnp.float32)
        sel_j = (dq == 8 * dk + j).astype(jnp.float32)
        mask_j = jax.lax.dot(sel_j, maskc,
                             preferred_element_type=jnp.float32)
        s_box = s_box + jnp.sum(folded * mask_j)

    @pl.when(jnp.logical_and(bb == 0, t == 0))
    def _():
        bce_ref[...] = jnp.zeros_like(bce_ref)
        box_ref[...] = jnp.zeros_like(box_ref)

    bce_ref[...] += jnp.broadcast_to(s_bce, (1, 1))
    box_ref[...] += jnp.broadcast_to(s_box, (1, 1))


def _tc_main(op4, gt4o, bpv, bgv):
    bbox_specs = [
        pl.BlockSpec((KSUB * 4, 128),
                     (lambda j: lambda bb, t: ((8 * bb + j) * 9 + t, 0))(j))
        for j in range(8)
    ]
    return pl.pallas_call(
        _tc_body,
        grid=(4, KT),
        in_specs=[
            pl.BlockSpec((OBJ_BLK, 128), lambda bb, t: (KT * bb + t, 0)),
            pl.BlockSpec((OBJ_BLK, 128), lambda bb, t: (KT * bb + t, 0)),
            pl.BlockSpec((KSUB * 8, 128), lambda bb, t: (9 * bb + t, 0)),
        ] + bbox_specs + bbox_specs,
        out_specs=[pl.BlockSpec((1, 1), lambda bb, t: (0, 0))] * 2,
        out_shape=[jax.ShapeDtypeStruct((1, 1), jnp.float32)] * 2,
    )(op4, gt4o, gt4o, *([bpv] * 8), *([bgv] * 8))


NW = 32
CHUNK_ROWS = 128
SC_ROWS = RPB - TC_FRAC
N_CHUNKS = SC_ROWS // CHUNK_ROWS
SC_K = KBLK - TC_K

_sc_mesh = plsc.VectorSubcoreMesh(core_axis_name="c", subcore_axis_name="s")


@functools.partial(
    pl.kernel,
    mesh=_sc_mesh,
    out_type=jax.ShapeDtypeStruct((NW, 16), jnp.float32),
    scratch_types=[
        pltpu.VMEM((SC_K, 128), jnp.float32),
        pltpu.VMEM((2, CHUNK_ROWS, 128), jnp.float32),
        pltpu.VMEM((2, CHUNK_ROWS, 128), jnp.float32),
        pltpu.VMEM((16,), jnp.float32),
        pltpu.SemaphoreType.DMA,
        pltpu.SemaphoreType.DMA,
        pltpu.SemaphoreType.DMA,
    ],
)
def _sc_bbox(bp_hbm, bg_hbm, gt4_hbm, out_hbm,
             mask_v, bp_v, bg_v, acc_v, sem_m, sem_p, sem_g):
    cid = lax.axis_index("c")
    sid = lax.axis_index("s")
    w = sid * 2 + cid
    bb = w // 8
    bi = w % 8

    pltpu.sync_copy(gt4_hbm.at[bb, pl.ds(TC_K, SC_K), bi, :], mask_v)

    row0 = w * RPB + TC_FRAC

    def chunk_start(step, buf):
        r = row0 + step * CHUNK_ROWS
        cp = pltpu.async_copy(bp_hbm.at[pl.ds(r, CHUNK_ROWS), :],
                              bp_v.at[buf], sem_p)
        cg = pltpu.async_copy(bg_hbm.at[pl.ds(r, CHUNK_ROWS), :],
                              bg_v.at[buf], sem_g)
        return cp, cg

    def chunk_sum(step, buf):
        k0 = step * (CHUNK_ROWS // 4)

        def kk_body(kk, acc):
            def a_body(ai, acc2):
                a0 = ai * 16
                m = mask_v[k0 + kk, pl.ds(a0, 16)]

                def c_body(cc, acc3):
                    r = kk * 4 + cc
                    p = bp_v[buf, r, pl.ds(a0, 16)]
                    g = bg_v[buf, r, pl.ds(a0, 16)]
                    d = p - g
                    a = jnp.abs(d)
                    cl = jnp.minimum(a, 1.0)
                    per2 = cl * (a + a - cl)
                    return acc3 + per2 * m

                return lax.fori_loop(0, 4, c_body, acc2)

            return lax.fori_loop(0, 8, a_body, acc)

        return lax.fori_loop(0, CHUNK_ROWS // 4, kk_body,
                             jnp.zeros((16,), jnp.float32))

    total = jnp.zeros((16,), jnp.float32)
    cp, cg = chunk_start(0, 0)
    for step in range(N_CHUNKS):
        cp.wait()
        cg.wait()
        if step + 1 < N_CHUNKS:
            cp, cg = chunk_start(step + 1, (step + 1) % 2)
        total = total + chunk_sum(step, step % 2)

    acc_v[...] = total
    pltpu.sync_copy(acc_v, out_hbm.at[w])


def _combine_body(scal_ref, part_ref, o1_ref, o2_ref, o3_ref):
    lam_o = scal_ref[0, 0]
    lam_b = scal_ref[0, 1]
    bce = scal_ref[0, 2]
    box_tc = scal_ref[0, 3]
    o1 = lam_o * bce * (1.0 / BN)
    o2 = lam_b * 0.5 * (jnp.sum(part_ref[...]) + box_tc)
    o1_ref[...] = jnp.broadcast_to(o1, (1, 1))
    o2_ref[...] = jnp.broadcast_to(o2, (1, 1))
    o3_ref[...] = jnp.broadcast_to(o1 + o2, (1, 1))


def _combine(scal, parts):
    return pl.pallas_call(
        _combine_body,
        in_specs=[
            pl.BlockSpec(memory_space=pltpu.SMEM),
            pl.BlockSpec((NW, 16), lambda: (0, 0)),
        ],
        out_specs=[pl.BlockSpec((1, 1), lambda: (0, 0))] * 3,
        out_shape=[jax.ShapeDtypeStruct((1, 1), jnp.float32)] * 3,
    )(scal, parts)


def kernel(objectness_pred, bbox_pred, objectness_gt, bbox_gt,
           lambda_rpn_objectness, lambda_rpn_bbox):
    bp_view = (bbox_pred.reshape(B, KBLK, 128, 4)
               .transpose(0, 1, 3, 2).reshape(ROWS, 128))
    bg_view = (bbox_gt.reshape(B, KBLK, 128, 4)
               .transpose(0, 1, 3, 2).reshape(ROWS, 128))
    gt4_view = (objectness_gt.reshape(4, 8, KBLK, 128)
                .transpose(0, 2, 1, 3))
    gt_obj = gt4_view.reshape(B * KBLK, 128)
    op_obj = (objectness_pred.reshape(4, 8, KBLK, 128)
              .transpose(0, 2, 1, 3).reshape(B * KBLK, 128))

    parts = _sc_bbox(bp_view, bg_view, gt4_view)
    bce, box_tc = _tc_main(op_obj, gt_obj, bp_view, bg_view)
    lam = jnp.stack([jnp.asarray(lambda_rpn_objectness, jnp.float32),
                     jnp.asarray(lambda_rpn_bbox, jnp.float32),
                     bce.reshape(()), box_tc.reshape(())]).reshape(1, 4)
    o1, o2, o3 = _combine(lam, parts)
    return (o1.reshape(()), o2.reshape(()), o3.reshape(()))
